# baseline (speedup 1.0000x reference)
import jax
import jax.numpy as jnp
from jax.experimental import pallas as pl

N_INNER = 2
N_ROUNDS = 2


def _mlp(params, prefix, x):
    i = 0
    while f"{prefix}_W{i}" in params:
        x = x @ params[f"{prefix}_W{i}"] + params[f"{prefix}_b{i}"]
        if f"{prefix}_W{i + 1}" in params:
            x = jax.nn.relu(x)
        i += 1
    return x


def _seg_softmax(scores, seg, num):
    m = jax.ops.segment_max(scores, seg, num_segments=num)
    m = jnp.where(jnp.isneginf(m), 0.0, m)
    ex = jnp.exp(scores - m[seg])
    s = jax.ops.segment_sum(ex, seg, num_segments=num)
    return ex / (s[seg] + 1e-16)


def _attention(params, x_cat, edge_index, e_cat, u_cat, batch):
    src, dest = edge_index[0], edge_index[1]
    n_nodes = x_cat.shape[0]
    n_graphs = u_cat.shape[0]
    edge_in = jnp.concatenate([x_cat[dest] - x_cat[src], e_cat, u_cat[batch[src]]], axis=1)
    a = _mlp(params, "att_a", edge_in)
    v = _mlp(params, "att_v", edge_in)
    attn = _seg_softmax(a, dest, n_nodes)
    msg = attn * v
    agg_sum = jax.ops.segment_sum(msg, dest, num_segments=n_nodes)
    agg_max = jax.ops.segment_max(msg, dest, num_segments=n_nodes)
    agg_max = jnp.where(jnp.isneginf(agg_max), 0.0, agg_max)
    e_h = v
    x_h = _mlp(params, "node", jnp.concatenate([x_cat, agg_sum, agg_max, u_cat[batch]], axis=1))
    cnt = jnp.maximum(jax.ops.segment_sum(jnp.ones((n_nodes,), jnp.float32), batch, num_segments=n_graphs), 1.0)
    g_mean = jax.ops.segment_sum(x_h, batch, num_segments=n_graphs) / cnt[:, None]
    g_max = jax.ops.segment_max(x_h, batch, num_segments=n_graphs)
    g_max = jnp.where(jnp.isneginf(g_max), 0.0, g_max)
    u_h = _mlp(params, "glob", jnp.concatenate([g_mean, g_max, u_cat], axis=1))
    return x_h, e_h, u_h


def kernel(x1, edge_index1, e1, u1, batch1, x2, edge_index2, e2, u2, batch2, params):
    def encode(x, e, u):
        return _mlp(params, "enc_x", x), _mlp(params, "enc_e", e), _mlp(params, "enc_u", u)

    def processing(x, x_h, edge_index, e, e_h, u, u_h, batch, shared):
        src = edge_index[0]
        for _ in range(N_INNER):
            x_cat = jnp.concatenate([x, x_h, shared[batch]], axis=1)
            e_cat = jnp.concatenate([e, e_h, shared[batch[src]]], axis=1)
            u_cat = jnp.concatenate([u, u_h, shared], axis=1)
            x_h, e_h, u_h = _attention(params, x_cat, edge_index, e_cat, u_cat, batch)
        return x_h, e_h, u_h

    x1h, e1h, u1h = encode(x1, e1, u1)
    x2h, e2h, u2h = encode(x2, e2, u2)
    outs = []
    for _ in range(N_ROUNDS):
        x1h, e1h, u1h = processing(x1, x1h, edge_index1, e1, e1h, u1, u1h, batch1, u2h)
        x2h, e2h, u2h = processing(x2, x2h, edge_index2, e2, e2h, u2, u2h, batch2, u1h)
        outs.append(_mlp(params, "dec", u2h))
    return jnp.stack(outs)


# speedup vs baseline: 3.9838x; 3.9838x over previous
import functools

import jax
import jax.numpy as jnp
from jax.experimental import pallas as pl
from jax.experimental.pallas import tpu as pltpu

N_INNER = 2
N_ROUNDS = 2
_dot = functools.partial(jnp.matmul, precision=jax.lax.Precision.HIGHEST)
N = 10000
E = 320000
B = 16
H = 32
NB = 2000
EB = 2048
EP = 327680
ER = EP // 128
NP = 10240
CR = 32
NEG = -3.0e38


def _full(shape):
    nd = len(shape)
    return pl.BlockSpec(shape, lambda i: (0,) * nd)


def _rows(bs, ncols):
    return pl.BlockSpec((bs, ncols), lambda i: (i, 0))


def _prep_body(x_ref, xh_ref, oh_ref, u_ref, uh_ref, sh_ref,
               wx_ref, wxh_ref, wsh_ref, wshe_ref, wu_ref, b0_ref,
               p_ref, q_ref):
    ucat = jnp.concatenate([u_ref[...], uh_ref[...], sh_ref[...]], axis=1)
    G = _dot(sh_ref[...], wshe_ref[...]) + _dot(ucat, wu_ref[...]) + b0_ref[...]
    shB = _dot(oh_ref[...], sh_ref[...])
    R = _dot(x_ref[...], wx_ref[...]) + _dot(xh_ref[...], wxh_ref[...]) + _dot(shB, wsh_ref[...])
    p_ref[...] = R
    q_ref[...] = _dot(oh_ref[...], G) - R


def _prep_call(x, xh, oh, u, uh, sh, wx, wxh, wsh, wshe, wu, b0):
    grid = (N // NB,)
    return pl.pallas_call(
        _prep_body,
        grid=grid,
        in_specs=[_rows(NB, 128), _rows(NB, H), _rows(NB, B),
                  _full((B, 16)), _full((B, H)), _full((B, H)),
                  _full((128, 128)), _full((H, 128)), _full((H, 128)),
                  _full((H, 128)), _full((80, 128)), _full((1, 128))],
        out_specs=[_rows(NB, 128), _rows(NB, 128)],
        out_shape=[jax.ShapeDtypeStruct((N, 128), jnp.float32),
                   jax.ShapeDtypeStruct((N, 128), jnp.float32)],
    )(x, xh, oh, u, uh, sh, wx, wxh, wsh, wshe, wu, b0)


def _edge_body(h0_ref, e_ref, eh_ref, wee_ref, w1a_ref, b1a_ref,
               w1v_ref, b1v_ref, v_ref, exT_ref, exvT_ref):
    ef = jnp.concatenate([e_ref[...], eh_ref[...]], axis=1)
    h = h0_ref[...] + _dot(ef, wee_ref[...])
    ha = jax.nn.relu(h[:, :64])
    hv = jax.nn.relu(h[:, 64:])
    a = _dot(ha, w1a_ref[...]) + b1a_ref[...]
    v = _dot(hv, w1v_ref[...]) + b1v_ref[...]
    ex = jnp.exp(a)
    v_ref[...] = v
    exT_ref[...] = ex.T.reshape(H, EB // 128, 128)
    exvT_ref[...] = (ex * v).T.reshape(H, EB // 128, 128)


def _edge_call(h0, e, eh, wee, w1a, b1a, w1v, b1v):
    grid = (EP // EB,)
    return pl.pallas_call(
        _edge_body,
        grid=grid,
        in_specs=[_rows(EB, 128), _rows(EB, 16), _rows(EB, H),
                  _full((48, 128)), _full((64, H)), _full((1, H)),
                  _full((64, H)), _full((1, H))],
        out_specs=[_rows(EB, H),
                   pl.BlockSpec((H, EB // 128, 128), lambda i: (0, i, 0)),
                   pl.BlockSpec((H, EB // 128, 128), lambda i: (0, i, 0))],
        out_shape=[jax.ShapeDtypeStruct((EP, H), jnp.float32),
                   jax.ShapeDtypeStruct((H, ER, 128), jnp.float32),
                   jax.ShapeDtypeStruct((H, ER, 128), jnp.float32)],
    )(h0, e, eh, wee, w1a, b1a, w1v, b1v)


def _node_body(x_ref, xh_ref, oh_ref, s_ref, ws_ref, wm_ref,
               u_ref, uh_ref, sh_ref,
               wnx_ref, wnxh_ref, wnsh_ref, wns_ref, wnm_ref, wnu_ref,
               b0_ref, w1_ref, b1_ref, out_ref):
    s = s_ref[...]
    inv = 1.0 / (s + 1e-16)
    agg_sum = ws_ref[...] * inv
    wm = wm_ref[...]
    agg_max = jnp.where(wm < NEG, 0.0, wm * inv)
    ucat = jnp.concatenate([u_ref[...], uh_ref[...], sh_ref[...]], axis=1)
    Gn = _dot(ucat, wnu_ref[...]) + b0_ref[...]
    shB = _dot(oh_ref[...], sh_ref[...])
    hid = (_dot(x_ref[...], wnx_ref[...]) + _dot(xh_ref[...], wnxh_ref[...])
           + _dot(shB, wnsh_ref[...]) + _dot(agg_sum, wns_ref[...])
           + _dot(agg_max, wnm_ref[...]) + _dot(oh_ref[...], Gn))
    out_ref[...] = _dot(jax.nn.relu(hid), w1_ref[...]) + b1_ref[...]


def _node_call(x, xh, oh, s, ws, wm, u, uh, sh,
               wnx, wnxh, wnsh, wns, wnm, wnu, b0, w1, b1):
    grid = (N // NB,)
    return pl.pallas_call(
        _node_body,
        grid=grid,
        in_specs=[_rows(NB, 128), _rows(NB, H), _rows(NB, B),
                  _rows(NB, H), _rows(NB, H), _rows(NB, H),
                  _full((B, 16)), _full((B, H)), _full((B, H)),
                  _full((128, 64)), _full((H, 64)), _full((H, 64)),
                  _full((H, 64)), _full((H, 64)), _full((80, 64)),
                  _full((1, 64)), _full((64, H)), _full((1, H))],
        out_specs=[_rows(NB, H)],
        out_shape=[jax.ShapeDtypeStruct((N, H), jnp.float32)],
    )(x, xh, oh, s, ws, wm, u, uh, sh, wnx, wnxh, wnsh, wns, wnm, wnu, b0, w1, b1)[0]


def _graph_body(xh_ref, oh_ref, u_ref, uh_ref, sh_ref,
                wg0_ref, bg0_ref, wg1_ref, bg1_ref, out_ref):
    xh = xh_ref[...]
    oh = oh_ref[...]
    gsum = jax.lax.dot_general(oh, xh, (((0,), (0,)), ((), ())), precision=jax.lax.Precision.HIGHEST)
    cnt = jnp.maximum(jnp.sum(oh, axis=0), 1.0)
    gmean = gsum / cnt[:, None]
    rows = []
    for g in range(B):
        m = jnp.max(jnp.where(oh[:, g:g + 1] > 0.5, xh, NEG), axis=0)
        rows.append(jnp.where(m < NEG * 0.5, 0.0, m))
    gmax = jnp.stack(rows, axis=0)
    ucat = jnp.concatenate([u_ref[...], uh_ref[...], sh_ref[...]], axis=1)
    inp = jnp.concatenate([gmean, gmax, ucat], axis=1)
    out_ref[...] = _dot(jax.nn.relu(_dot(inp, wg0_ref[...]) + bg0_ref[...]), wg1_ref[...]) + bg1_ref[...]


def _graph_call(xh, oh, u, uh, sh, wg0, bg0, wg1, bg1):
    return pl.pallas_call(
        _graph_body,
        grid=(1,),
        in_specs=[_full((N, H)), _full((N, B)),
                  _full((B, 16)), _full((B, H)), _full((B, H)),
                  _full((144, 64)), _full((1, 64)), _full((64, H)), _full((1, H))],
        out_specs=[_full((B, H))],
        out_shape=[jax.ShapeDtypeStruct((B, H), jnp.float32)],
    )(xh, oh, u, uh, sh, wg0, bg0, wg1, bg1)[0]


def _mlp_body(x_ref, w0_ref, b0_ref, w1_ref, b1_ref, out_ref):
    h = jax.nn.relu(_dot(x_ref[...], w0_ref[...]) + b0_ref[...])
    out_ref[...] = _dot(h, w1_ref[...]) + b1_ref[...]


def _mlp_call(x, w0, b0, w1, b1, bs):
    n, fi = x.shape
    fh = w0.shape[1]
    fo = w1.shape[1]
    grid = (n // bs,)
    return pl.pallas_call(
        _mlp_body,
        grid=grid,
        in_specs=[_rows(bs, fi), _full((fi, fh)), _full((1, fh)),
                  _full((fh, fo)), _full((1, fo))],
        out_specs=[_rows(bs, fo)],
        out_shape=[jax.ShapeDtypeStruct((n, fo), jnp.float32)],
    )(x, w0, b0, w1, b1)[0]


from jax import lax
from jax.experimental.pallas import tpu_sc as plsc

C2 = 128
C4 = 4000
NW = 32


def _sc_gather_call(P, Q, dest, src):
    mesh = plsc.VectorSubcoreMesh(core_axis_name="c", subcore_axis_name="s")
    epw = EP // NW

    @functools.partial(
        pl.kernel, mesh=mesh,
        compiler_params=pltpu.CompilerParams(needs_layout_passes=False),
        out_type=jax.ShapeDtypeStruct((EP, 128), jnp.float32),
        scratch_types=[pltpu.VMEM((C2,), jnp.int32),
                       pltpu.VMEM((C2,), jnp.int32),
                       pltpu.VMEM((C2, 128), jnp.float32),
                       pltpu.VMEM((C2, 128), jnp.float32),
                       pltpu.SemaphoreType.DMA,
                       pltpu.SemaphoreType.DMA],
    )
    def k(p_hbm, q_hbm, dest_hbm, src_hbm, out_hbm, didx, sidx, bufp, bufq, sem1, sem2):
        wid = lax.axis_index("s") * 2 + lax.axis_index("c")
        base = wid * epw

        def chunk(ci, _):
            off = base + ci * C2
            pltpu.sync_copy(dest_hbm.at[pl.ds(off, C2)], didx)
            pltpu.sync_copy(src_hbm.at[pl.ds(off, C2)], sidx)
            cp1 = pltpu.async_copy(p_hbm.at[didx], bufp, sem1)
            cp2 = pltpu.async_copy(q_hbm.at[sidx], bufq, sem2)
            cp1.wait()
            cp2.wait()

            def addr(r, _):
                for c8 in range(8):
                    bufp[r, pl.ds(c8 * 16, 16)] = (bufp[r, pl.ds(c8 * 16, 16)]
                                                   + bufq[r, pl.ds(c8 * 16, 16)])
                return 0

            lax.fori_loop(0, C2, addr, 0)
            pltpu.sync_copy(bufp, out_hbm.at[pl.ds(off, C2)])
            return 0

        lax.fori_loop(0, epw // C2, chunk, 0)

    return k(P, Q, dest, src)


def _sc_segment_call(exT, exvT, dest):
    mesh = plsc.VectorSubcoreMesh(core_axis_name="c", subcore_axis_name="s")
    AR = NP // 128

    @functools.partial(
        pl.kernel, mesh=mesh,
        compiler_params=pltpu.CompilerParams(needs_layout_passes=False),
        out_type=[jax.ShapeDtypeStruct((H, AR, 128), jnp.float32),
                  jax.ShapeDtypeStruct((H, AR, 128), jnp.float32),
                  jax.ShapeDtypeStruct((H, AR, 128), jnp.float32)],
        scratch_types=[pltpu.VMEM((CR * 128,), jnp.int32),
                       pltpu.VMEM((CR, 128), jnp.float32),
                       pltpu.VMEM((CR, 128), jnp.float32),
                       pltpu.VMEM((AR, 128), jnp.float32),
                       pltpu.VMEM((AR, 128), jnp.float32),
                       pltpu.VMEM((AR, 128), jnp.float32)],
    )
    def k(exT_hbm, exvT_hbm, dest_hbm, outs_hbm, outw_hbm, outm_hbm,
          didx, exb, exvb, sacc, wacc, macc):
        wid = lax.axis_index("s") * 2 + lax.axis_index("c")

        def init(i, _):
            def initc(c8, _):
                sacc[i, pl.ds(c8 * 16, 16)] = jnp.zeros((16,), jnp.float32)
                wacc[i, pl.ds(c8 * 16, 16)] = jnp.zeros((16,), jnp.float32)
                macc[i, pl.ds(c8 * 16, 16)] = jnp.full((16,), -jnp.inf, jnp.float32)
                return 0
            return lax.fori_loop(0, 8, initc, 0, unroll=True)

        lax.fori_loop(0, AR, init, 0)

        def chunk(ci, _):
            pltpu.sync_copy(dest_hbm.at[pl.ds(ci * CR * 128, CR * 128)], didx)
            pltpu.sync_copy(exT_hbm.at[wid, pl.ds(ci * CR, CR)], exb)
            pltpu.sync_copy(exvT_hbm.at[wid, pl.ds(ci * CR, CR)], exvb)

            def grp(r, _):
                for c8 in range(8):
                    idx = didx[pl.ds(r * 128 + c8 * 16, 16)]
                    ir = jax.lax.shift_right_logical(idx, 7)
                    ic = jax.lax.bitwise_and(idx, 127)
                    xe = exb[r, pl.ds(c8 * 16, 16)]
                    xw = exvb[r, pl.ds(c8 * 16, 16)]
                    plsc.addupdate_scatter(sacc, [ir, ic], xe)
                    plsc.addupdate_scatter(wacc, [ir, ic], xw)

                    def rmw(carry, ir=ir, ic=ic, xw=xw):
                        cur = plsc.load_gather(macc, [ir, ic])
                        lost = cur < xw
                        plsc.store_scatter(macc, [ir, ic], jnp.maximum(cur, xw),
                                           mask=lost)
                        cur2 = plsc.load_gather(macc, [ir, ic])
                        return jnp.max((cur2 < xw).astype(jnp.int32)) > 0

                    lax.while_loop(lambda c_: c_, rmw, jnp.bool_(True))
                return 0

            lax.fori_loop(0, CR, grp, 0)
            return 0

        lax.fori_loop(0, ER // CR, chunk, 0)
        pltpu.sync_copy(sacc, outs_hbm.at[wid])
        pltpu.sync_copy(wacc, outw_hbm.at[wid])
        pltpu.sync_copy(macc, outm_hbm.at[wid])

    return k(exT, exvT, dest)


def _att_weights(params):
    w0 = jnp.concatenate([params["att_a_W0"], params["att_v_W0"]], axis=1)
    b0 = jnp.concatenate([params["att_a_b0"], params["att_v_b0"]])[None, :]
    wn0 = params["node_W0"]
    return dict(
        wx=w0[0:128], wxh=w0[128:160], wsh=w0[160:192],
        wee=w0[192:240], wshe=w0[240:272], wu=w0[272:352], b0=b0,
        w1a=params["att_a_W1"], b1a=params["att_a_b1"][None, :],
        w1v=params["att_v_W1"], b1v=params["att_v_b1"][None, :],
        wnx=wn0[0:128], wnxh=wn0[128:160], wnsh=wn0[160:192],
        wns=wn0[192:224], wnm=wn0[224:256], wnu=wn0[256:336],
        b0n=params["node_b0"][None, :],
        wn1=params["node_W1"], b1n=params["node_b1"][None, :],
        wg0=params["glob_W0"], bg0=params["glob_b0"][None, :],
        wg1=params["glob_W1"], bg1=params["glob_b1"][None, :],
    )


def _attention_block(W, x, e, oh, dest_g, src_g, dest_s, xh, eh, u, uh, shared):
    P, Q = _prep_call(x, xh, oh, u, uh, shared,
                      W["wx"], W["wxh"], W["wsh"], W["wshe"], W["wu"], W["b0"])
    h0 = _sc_gather_call(P, Q, dest_g, src_g)
    v, exT, exvT = _edge_call(h0, e, eh, W["wee"], W["w1a"], W["b1a"],
                              W["w1v"], W["b1v"])
    sT, wsT, wmT = _sc_segment_call(exT, exvT, dest_s)
    s = sT.reshape(H, NP)[:, :N].T
    ws = wsT.reshape(H, NP)[:, :N].T
    wm = wmT.reshape(H, NP)[:, :N].T
    xh2 = _node_call(x, xh, oh, s, ws, wm, u, uh, shared,
                     W["wnx"], W["wnxh"], W["wnsh"], W["wns"], W["wnm"],
                     W["wnu"], W["b0n"], W["wn1"], W["b1n"])
    uh2 = _graph_call(xh2, oh, u, uh, shared,
                      W["wg0"], W["bg0"], W["wg1"], W["bg1"])
    return xh2, v, uh2


def kernel(x1, edge_index1, e1, u1, batch1, x2, edge_index2, e2, u2, batch2, params):
    W = _att_weights(params)
    oh1 = (batch1[:, None] == jnp.arange(B, dtype=batch1.dtype)[None, :]).astype(jnp.float32)
    oh2 = (batch2[:, None] == jnp.arange(B, dtype=batch2.dtype)[None, :]).astype(jnp.float32)
    npad = EP - E
    spread = jnp.arange(npad, dtype=jnp.int32) % N
    sent = N + jnp.arange(npad, dtype=jnp.int32) % (NP - N)
    src1g = jnp.concatenate([edge_index1[0], spread])
    dst1g = jnp.concatenate([edge_index1[1], spread])
    dst1s = jnp.concatenate([edge_index1[1], sent])
    src2g = jnp.concatenate([edge_index2[0], spread])
    dst2g = jnp.concatenate([edge_index2[1], spread])
    dst2s = jnp.concatenate([edge_index2[1], sent])
    e1p = jnp.concatenate([e1, jnp.zeros((npad, e1.shape[1]), jnp.float32)])
    e2p = jnp.concatenate([e2, jnp.zeros((npad, e2.shape[1]), jnp.float32)])

    x1h = _mlp_call(x1, params["enc_x_W0"], params["enc_x_b0"][None, :],
                    params["enc_x_W1"], params["enc_x_b1"][None, :], NB)
    x2h = _mlp_call(x2, params["enc_x_W0"], params["enc_x_b0"][None, :],
                    params["enc_x_W1"], params["enc_x_b1"][None, :], NB)
    e1h = _mlp_call(e1p, params["enc_e_W0"], params["enc_e_b0"][None, :],
                    params["enc_e_W1"], params["enc_e_b1"][None, :], EB)
    e2h = _mlp_call(e2p, params["enc_e_W0"], params["enc_e_b0"][None, :],
                    params["enc_e_W1"], params["enc_e_b1"][None, :], EB)
    u1h = _mlp_call(u1, params["enc_u_W0"], params["enc_u_b0"][None, :],
                    params["enc_u_W1"], params["enc_u_b1"][None, :], B)
    u2h = _mlp_call(u2, params["enc_u_W0"], params["enc_u_b0"][None, :],
                    params["enc_u_W1"], params["enc_u_b1"][None, :], B)

    outs = []
    for _ in range(N_ROUNDS):
        for _ in range(N_INNER):
            x1h, e1h, u1h = _attention_block(W, x1, e1p, oh1, dst1g, src1g,
                                             dst1s, x1h, e1h, u1, u1h, u2h)
        for _ in range(N_INNER):
            x2h, e2h, u2h = _attention_block(W, x2, e2p, oh2, dst2g, src2g,
                                             dst2s, x2h, e2h, u2, u2h, u1h)
        outs.append(_mlp_call(u2h, params["dec_W0"], params["dec_b0"][None, :],
                              params["dec_W1"], params["dec_b1"][None, :], B))
    return jnp.stack(outs)
